# baseline (speedup 1.0000x reference)
import functools

import jax
import jax.numpy as jnp
from jax import lax
from jax.experimental import pallas as pl
from jax.experimental.pallas import tpu as pltpu
from jax.experimental.pallas import tpu_sc as plsc

N = 10000
E = 320000
D = 128
C = 40
B = 64

NPAD = 10240
CHUNK = 128
NW = 32
EPAD = 327680
NCH = EPAD // CHUNK
CH_PER_TILE = NCH // NW
EDGES_PER_TILE = CH_PER_TILE * CHUNK
ROWS_PER_TILE = NPAD // 16
RB = 10
RBLK = N // RB

NTILE = NPAD // NW
TRASH = NTILE
SROWS = NTILE + 16
BMUL = 6554
BSH = 21
BALIGN = 1024
SSZ = EPAD + NW * BALIGN + CHUNK
TRASH_POS = SSZ - CHUNK


def _zero_f32_block(ref, rows, cols):
  groups = cols // 16

  def body(t, carry):
    i = t // groups
    g = t % groups
    ref[i, pl.ds(g * 16, 16)] = jnp.zeros((16,), jnp.float32)
    return carry

  lax.fori_loop(0, rows * groups, body, None)


def _fill_f32_1d(ref, n, value):
  def body(t, carry):
    ref[pl.ds(t * 16, 16)] = jnp.full((16,), value, jnp.float32)
    return carry

  lax.fori_loop(0, n // 16, body, None)


def _sc_deg_hist(dstc):
  mesh = plsc.VectorSubcoreMesh(core_axis_name="c", subcore_axis_name="s")

  @functools.partial(
      pl.kernel,
      out_type=(
          jax.ShapeDtypeStruct((2 * NPAD,), jnp.float32),
          jax.ShapeDtypeStruct((NW, NW), jnp.int32),
      ),
      mesh=mesh,
      scratch_types=[
          pltpu.VMEM((CH_PER_TILE, CHUNK), jnp.int32),
          pltpu.VMEM((CHUNK,), jnp.float32),
          pltpu.VMEM((ROWS_PER_TILE,), jnp.float32),
          pltpu.VMEM((NW,), jnp.int32),
          pltpu.SMEM((NW,), jnp.int32),
          pltpu.VMEM_SHARED((NPAD,), jnp.float32),
      ],
  )
  def k(dstc_hbm, deg_out, hist_out, idxd_v, ones_v, zeros_v, hist_v, hist_sm,
        deg_sh):
    c = lax.axis_index("c")
    s = lax.axis_index("s")
    wid = s * 2 + c
    _fill_f32_1d(ones_v, CHUNK, 1.0)
    _fill_f32_1d(zeros_v, ROWS_PER_TILE, 0.0)
    for b in range(NW):
      hist_sm[b] = jnp.int32(0)
    pltpu.sync_copy(zeros_v, deg_sh.at[pl.ds(s * ROWS_PER_TILE, ROWS_PER_TILE)])
    plsc.subcore_barrier()
    pltpu.sync_copy(dstc_hbm.at[pl.ds(wid * CH_PER_TILE, CH_PER_TILE)], idxd_v)

    def body(j, carry):
      pltpu.sync_copy(ones_v, deg_sh.at[idxd_v.at[j]], add=True)
      return carry

    lax.fori_loop(0, CH_PER_TILE, body, None)

    def hbody(t, carry):
      j = t // (CHUNK // 16)
      g = t % (CHUNK // 16)
      d = idxd_v[j, pl.ds(g * 16, 16)]
      bkt = lax.shift_right_logical(d * BMUL, BSH)
      for l in range(16):
        bl_ = bkt[l]
        hist_sm[bl_] = hist_sm[bl_] + 1
      return carry

    lax.fori_loop(0, CH_PER_TILE * (CHUNK // 16), hbody, None)

    lanes = lax.iota(jnp.int32, 16)
    for g in range(NW // 16):
      vec = jnp.zeros((16,), jnp.int32)
      for l in range(16):
        vec = jnp.where(lanes == l, hist_sm[g * 16 + l], vec)
      hist_v[pl.ds(g * 16, 16)] = vec

    plsc.subcore_barrier()
    pltpu.sync_copy(
        deg_sh.at[pl.ds(s * ROWS_PER_TILE, ROWS_PER_TILE)],
        deg_out.at[pl.ds(c * NPAD + s * ROWS_PER_TILE, ROWS_PER_TILE)],
    )
    pltpu.sync_copy(hist_v, hist_out.at[wid])

  return k(dstc)


def _sc_prefix(hist):
  mesh = plsc.VectorSubcoreMesh(core_axis_name="c", subcore_axis_name="s")

  @functools.partial(
      pl.kernel,
      out_type=(
          jax.ShapeDtypeStruct((NW, NW), jnp.int32),
          jax.ShapeDtypeStruct((NW, 16), jnp.int32),
      ),
      mesh=mesh,
      scratch_types=[
          pltpu.VMEM((NW, NW), jnp.int32),
          pltpu.VMEM((NW, NW), jnp.int32),
          pltpu.VMEM((NW, 16), jnp.int32),
      ],
  )
  def k(hist_hbm, off_out, metab_out, hist_v, off_v, metab_v):
    c = lax.axis_index("c")
    s = lax.axis_index("s")
    wid = s * 2 + c

    @pl.when(wid == 0)
    def _():
      pltpu.sync_copy(hist_hbm, hist_v)

      def tsum(g):
        def body(p, acc):
          return acc + hist_v[p, pl.ds(g * 16, 16)]

        return lax.fori_loop(0, NW, body, jnp.zeros((16,), jnp.int32))

      tot0 = tsum(0)
      tot1 = tsum(1)
      pad0 = lax.shift_left(
          lax.shift_right_logical(tot0 + (BALIGN - 1), 10), 10)
      pad1 = lax.shift_left(
          lax.shift_right_logical(tot1 + (BALIGN - 1), 10), 10)
      lanes = lax.iota(jnp.int32, 16)
      run = jnp.int32(0)
      base0 = jnp.zeros((16,), jnp.int32)
      for l in range(16):
        base0 = jnp.where(lanes == l, run, base0)
        run = run + pad0[l]
      base1 = jnp.zeros((16,), jnp.int32)
      for l in range(16):
        base1 = jnp.where(lanes == l, run, base1)
        run = run + pad1[l]

      def poff(p, run):
        run0, run1 = run
        off_v[p, pl.ds(0, 16)] = run0
        off_v[p, pl.ds(16, 16)] = run1
        return (run0 + hist_v[p, pl.ds(0, 16)],
                run1 + hist_v[p, pl.ds(16, 16)])

      lax.fori_loop(0, NW, poff, (base0, base1))

      for b in range(NW):
        g = b // 16
        l = b % 16
        base = (base0, base1)[g][l]
        tot = (tot0, tot1)[g][l]
        padded = (pad0, pad1)[g][l]
        row = jnp.where(lanes == 0, base, jnp.zeros((16,), jnp.int32))
        row = jnp.where(lanes == 1, tot, row)
        row = jnp.where(lanes == 2, lax.shift_right_logical(padded, 7), row)
        row = jnp.where(lanes == 3, padded - tot, row)
        metab_v[b, pl.ds(0, 16)] = row
      pltpu.sync_copy(off_v, off_out)
      pltpu.sync_copy(metab_v, metab_out)

  return k(hist)


def _sc_permute(srcc, dstc, off, metab):
  mesh = plsc.VectorSubcoreMesh(core_axis_name="c", subcore_axis_name="s")

  @functools.partial(
      pl.kernel,
      out_type=(
          jax.ShapeDtypeStruct((SSZ,), jnp.int32),
          jax.ShapeDtypeStruct((SSZ,), jnp.int32),
      ),
      mesh=mesh,
      scratch_types=[
          pltpu.VMEM((CH_PER_TILE, CHUNK), jnp.int32),
          pltpu.VMEM((CH_PER_TILE, CHUNK), jnp.int32),
          pltpu.VMEM((CH_PER_TILE, CHUNK), jnp.int32),
          pltpu.VMEM((CH_PER_TILE, CHUNK), jnp.int32),
          pltpu.VMEM((NW,), jnp.int32),
          pltpu.VMEM((16,), jnp.int32),
          pltpu.VMEM((BALIGN // CHUNK, CHUNK), jnp.int32),
          pltpu.VMEM((CHUNK,), jnp.int32),
          pltpu.SMEM((NW,), jnp.int32),
          [pltpu.SemaphoreType.DMA] * 8,
      ],
  )
  def k(srcc_hbm, dstc_hbm, off_hbm, metab_hbm, psrc_out, pdl_out,
        idxs_v, idxd_v, pos_v, dl_v, cur_v, mrow_v, padpos_v, padval_v,
        cur_sm, sems):
    c = lax.axis_index("c")
    s = lax.axis_index("s")
    wid = s * 2 + c
    base_ch = wid * CH_PER_TILE
    pltpu.sync_copy(srcc_hbm.at[pl.ds(base_ch, CH_PER_TILE)], idxs_v)
    pltpu.sync_copy(dstc_hbm.at[pl.ds(base_ch, CH_PER_TILE)], idxd_v)
    pltpu.sync_copy(off_hbm.at[wid], cur_v)
    pltpu.sync_copy(metab_hbm.at[wid], mrow_v)
    for g in range(NW // 16):
      cv = cur_v[pl.ds(g * 16, 16)]
      for l in range(16):
        cur_sm[g * 16 + l] = cv[l]

    lanes = lax.iota(jnp.int32, 16)

    def epass(t, carry):
      j = t // (CHUNK // 16)
      g = t % (CHUNK // 16)
      d = idxd_v[j, pl.ds(g * 16, 16)]
      bkt = lax.shift_right_logical(d * BMUL, BSH)
      dl_v[j, pl.ds(g * 16, 16)] = d - bkt * NTILE
      bls = [bkt[l] for l in range(16)]
      pvec = jnp.zeros((16,), jnp.int32)
      for l in range(16):
        p = cur_sm[bls[l]]
        cur_sm[bls[l]] = p + 1
        pvec = jnp.where(lanes == l, p, pvec)
      pos_v[j, pl.ds(g * 16, 16)] = pvec
      return carry

    lax.fori_loop(0, CH_PER_TILE * (CHUNK // 16), epass, None)

    @pl.loop(0, CH_PER_TILE, step=4)
    def _scat(j):
      descs = []
      for t in range(4):
        descs.append(pltpu.async_copy(
            idxs_v.at[j + t], psrc_out.at[pos_v.at[j + t]], sems[2 * t]))
        descs.append(pltpu.async_copy(
            dl_v.at[j + t], pdl_out.at[pos_v.at[j + t]], sems[2 * t + 1]))
      for dsc in descs:
        dsc.wait()

    mvec = mrow_v[pl.ds(0, 16)]
    bb = mvec[0] + mvec[1]
    padn = mvec[3]
    for g in range(CHUNK // 16):
      padval_v[pl.ds(g * 16, 16)] = jnp.zeros((16,), jnp.int32)
    for r in range(BALIGN // CHUNK):
      for g in range(CHUNK // 16):
        ids = lax.iota(jnp.int32, 16) + (r * CHUNK + g * 16)
        pos = jnp.where(ids < padn, bb + ids,
                        TRASH_POS + (ids & (CHUNK - 1)))
        padpos_v[r, pl.ds(g * 16, 16)] = pos
      pltpu.sync_copy(padval_v, psrc_out.at[padpos_v.at[r]])

    trash_row = jnp.full((16,), TRASH, jnp.int32)
    for g in range(CHUNK // 16):
      padval_v[pl.ds(g * 16, 16)] = trash_row
    for r in range(BALIGN // CHUNK):
      pltpu.sync_copy(padval_v, pdl_out.at[padpos_v.at[r]])

  return k(srcc, dstc, off, metab)


def _sc_consume(gs, psrc, pdl, metab):
  mesh = plsc.VectorSubcoreMesh(core_axis_name="c", subcore_axis_name="s")

  @functools.partial(
      pl.kernel,
      out_type=jax.ShapeDtypeStruct((NPAD * D,), jnp.float32),
      mesh=mesh,
      scratch_types=[
          pltpu.VMEM((SROWS * D,), jnp.float32),
          [pltpu.VMEM((CHUNK, D), jnp.float32)] * 4,
          pltpu.VMEM((8, CHUNK), jnp.int32),
          pltpu.VMEM((8, CHUNK), jnp.int32),
          pltpu.VMEM((16,), jnp.int32),
          [pltpu.SemaphoreType.DMA] * 4,
      ],
  )
  def k(gs_hbm, psrc_hbm, pdl_hbm, metab_hbm, out_hbm,
        s_local, bufs, sidx_blk, dl_blk, mrow_v, sems):
    c = lax.axis_index("c")
    s = lax.axis_index("s")
    wid = s * 2 + c
    pltpu.sync_copy(metab_hbm.at[wid], mrow_v)
    mvec = mrow_v[pl.ds(0, 16)]
    base = mvec[0]
    nch = mvec[2]
    _fill_f32_1d(s_local, SROWS * D, 0.0)

    brow = lax.shift_right_logical(base, 7)

    @pl.loop(0, nch, step=8)
    def _chunks(j):
      roff = pl.multiple_of(brow + j, 8)
      pltpu.sync_copy(psrc_hbm.at[pl.ds(roff, 8)], sidx_blk)
      pltpu.sync_copy(pdl_hbm.at[pl.ds(roff, 8)], dl_blk)
      for q in range(2):
        descs = [
            pltpu.async_copy(
                gs_hbm.at[sidx_blk.at[q * 4 + t]], bufs[t], sems[t])
            for t in range(4)
        ]
        for t in range(4):
          descs[t].wait()
          buf = bufs[t]

          def acc(t2, carry):
            vec = dl_blk[q * 4 + t, pl.ds(t2 * 16, 16)]
            drows = [pl.multiple_of(vec[l] * D, D) for l in range(16)]
            for g in range(D // 16):
              for l in range(16):
                plsc.addupdate(
                    s_local.at[pl.ds(drows[l] + g * 16, 16)],
                    buf[t2 * 16 + l, pl.ds(g * 16, 16)])
            return carry

          lax.fori_loop(0, CHUNK // 16, acc, None)

    pltpu.sync_copy(
        s_local.at[pl.ds(0, NTILE * D)],
        out_hbm.at[pl.ds(wid * NTILE * D, NTILE * D)])

  return k(gs, psrc, pdl, metab)


def _tc_dis(d0, d1):
  def body(d0_ref, d1_ref, out_ref):
    out_ref[...] = lax.rsqrt(d0_ref[...] + d1_ref[...] + 1.0)

  return pl.pallas_call(
      body,
      out_shape=jax.ShapeDtypeStruct((NPAD // 128, 128), jnp.float32),
  )(d0, d1)


def _tc_first(x, W, dis_col):
  def body(x_ref, w_ref, dis_ref, out_ref):
    g = jnp.dot(x_ref[...], w_ref[...], preferred_element_type=jnp.float32)
    out_ref[...] = dis_ref[...] * g

  return pl.pallas_call(
      body,
      grid=(RB,),
      in_specs=[
          pl.BlockSpec((RBLK, D), lambda r: (r, 0)),
          pl.BlockSpec((D, D), lambda r: (0, 0)),
          pl.BlockSpec((RBLK, 1), lambda r: (r, 0)),
      ],
      out_specs=pl.BlockSpec((RBLK, D), lambda r: (r, 0)),
      out_shape=jax.ShapeDtypeStruct((N, D), jnp.float32),
  )(x, W, dis_col)


def _tc_mid(s0, gsp, dis_col, bias, W):
  def body(s0_ref, gsp_ref, dis_ref, b_ref, w_ref, out_ref):
    agg = dis_ref[...] * (s0_ref[...] + gsp_ref[...])
    h = jnp.maximum(agg + b_ref[...], 0.0)
    g = jnp.dot(h, w_ref[...], preferred_element_type=jnp.float32)
    out_ref[...] = dis_ref[...] * g

  return pl.pallas_call(
      body,
      grid=(RB,),
      in_specs=[
          pl.BlockSpec((RBLK, D), lambda r: (r, 0)),
          pl.BlockSpec((RBLK, D), lambda r: (r, 0)),
          pl.BlockSpec((RBLK, 1), lambda r: (r, 0)),
          pl.BlockSpec((1, D), lambda r: (0, 0)),
          pl.BlockSpec((D, D), lambda r: (0, 0)),
      ],
      out_specs=pl.BlockSpec((RBLK, D), lambda r: (r, 0)),
      out_shape=jax.ShapeDtypeStruct((N, D), jnp.float32),
  )(s0, gsp, dis_col, bias, W)


def _tc_final(s0, gsp, dis_col, bias, batch3, Wl, bl):
  def body(s0_ref, gsp_ref, dis_ref, b_ref, bat_ref, wl_ref, bl_ref,
           out_ref, acc_ref, cnt_ref):
    r = pl.program_id(0)

    @pl.when(r == 0)
    def _():
      acc_ref[...] = jnp.zeros((B, D), jnp.float32)
      cnt_ref[...] = jnp.zeros((B, 128), jnp.float32)

    agg = dis_ref[...] * (s0_ref[...] + gsp_ref[...])
    h = agg + b_ref[...]
    bat = bat_ref[0]
    gid = lax.broadcasted_iota(jnp.int32, (B, RBLK), 0)
    p = jnp.where(bat == gid, 1.0, 0.0)
    acc_ref[...] += jnp.dot(p, h, preferred_element_type=jnp.float32)
    cnt_ref[...] += jnp.broadcast_to(
        jnp.sum(p, axis=1, keepdims=True), (B, 128))

    @pl.when(r == RB - 1)
    def _():
      x_g = acc_ref[...] / jnp.maximum(cnt_ref[...], 1.0)
      logits = (
          jnp.dot(x_g, wl_ref[...], preferred_element_type=jnp.float32)
          + bl_ref[...])
      m = jnp.max(logits, axis=-1, keepdims=True)
      z = logits - m
      lse = jnp.log(jnp.sum(jnp.exp(z), axis=-1, keepdims=True))
      out_ref[...] = z - lse

  return pl.pallas_call(
      body,
      grid=(RB,),
      in_specs=[
          pl.BlockSpec((RBLK, D), lambda r: (r, 0)),
          pl.BlockSpec((RBLK, D), lambda r: (r, 0)),
          pl.BlockSpec((RBLK, 1), lambda r: (r, 0)),
          pl.BlockSpec((1, D), lambda r: (0, 0)),
          pl.BlockSpec((1, 1, RBLK), lambda r: (r, 0, 0)),
          pl.BlockSpec((D, C), lambda r: (0, 0)),
          pl.BlockSpec((1, C), lambda r: (0, 0)),
      ],
      out_specs=pl.BlockSpec((B, C), lambda r: (0, 0)),
      out_shape=jax.ShapeDtypeStruct((B, C), jnp.float32),
      scratch_shapes=[
          pltpu.VMEM((B, D), jnp.float32),
          pltpu.VMEM((B, 128), jnp.float32),
      ],
  )(s0, gsp, dis_col, bias, batch3, Wl, bl)


def kernel(x, edge_index, batch, W1, b1, W2, b2, W3, b3, Wl, bl):
  src = edge_index[0]
  dst = edge_index[1]
  pad = EPAD - E
  srcc = jnp.concatenate([src, jnp.zeros((pad,), jnp.int32)]).reshape(
      NCH, CHUNK)
  dstc = jnp.concatenate([dst, jnp.full((pad,), N, jnp.int32)]).reshape(
      NCH, CHUNK)

  degp, hist = _sc_deg_hist(dstc)
  off, metab = _sc_prefix(hist)
  psrc, pdl = _sc_permute(srcc, dstc, off, metab)

  d0 = degp[:NPAD].reshape(NPAD // 128, 128)
  d1 = degp[NPAD:].reshape(NPAD // 128, 128)
  dis_col = _tc_dis(d0, d1).reshape(NPAD)[:N].reshape(N, 1)

  b1r = b1.reshape(1, D)
  b2r = b2.reshape(1, D)
  b3r = b3.reshape(1, D)
  blr = bl.reshape(1, C)
  batch3 = batch.reshape(RB, 1, RBLK)

  gs1 = _tc_first(x, W1, dis_col)
  psrc2 = psrc.reshape(SSZ // CHUNK, CHUNK)
  pdl2 = pdl.reshape(SSZ // CHUNK, CHUNK)
  s = _sc_consume(gs1, psrc2, pdl2, metab).reshape(NPAD, D)
  gs2 = _tc_mid(s[:N], gs1, dis_col, b1r, W2)
  s = _sc_consume(gs2, psrc2, pdl2, metab).reshape(NPAD, D)
  gs3 = _tc_mid(s[:N], gs2, dis_col, b2r, W3)
  s = _sc_consume(gs3, psrc2, pdl2, metab).reshape(NPAD, D)
  return _tc_final(s[:N], gs3, dis_col, b3r, batch3, Wl, blr)


# speedup vs baseline: 4.7604x; 4.7604x over previous
import functools

import jax
import jax.numpy as jnp
from jax import lax
from jax.experimental import pallas as pl
from jax.experimental.pallas import tpu as pltpu
from jax.experimental.pallas import tpu_sc as plsc

N = 10000
E = 320000
D = 128
C = 40
B = 64

NPAD = 10240
CHUNK = 128
NW = 32
EPAD = 327680
NCH = EPAD // CHUNK
CH_PER_TILE = NCH // NW
ROWS_PER_TILE = NPAD // 16
RB = 10
RBLK = N // RB


def _zero_f32_block(ref, rows, cols):
  groups = cols // 16

  def body(t, carry):
    i = t // groups
    g = t % groups
    ref[i, pl.ds(g * 16, 16)] = jnp.zeros((16,), jnp.float32)
    return carry

  lax.fori_loop(0, rows * groups, body, None)


def _fill_f32_1d(ref, n, value):
  def body(t, carry):
    ref[pl.ds(t * 16, 16)] = jnp.full((16,), value, jnp.float32)
    return carry

  lax.fori_loop(0, n // 16, body, None)


def _sc_degree(dstc):
  mesh = plsc.VectorSubcoreMesh(core_axis_name="c", subcore_axis_name="s")

  @functools.partial(
      pl.kernel,
      out_type=jax.ShapeDtypeStruct((2 * NPAD,), jnp.float32),
      mesh=mesh,
      scratch_types=[
          pltpu.VMEM((CH_PER_TILE, CHUNK), jnp.int32),
          pltpu.VMEM((CHUNK,), jnp.float32),
          pltpu.VMEM((ROWS_PER_TILE,), jnp.float32),
          pltpu.VMEM_SHARED((NPAD,), jnp.float32),
      ],
  )
  def k(dstc_hbm, out_hbm, idxd_v, ones_v, zeros_v, deg_sh):
    c = lax.axis_index("c")
    s = lax.axis_index("s")
    wid = s * 2 + c
    _fill_f32_1d(ones_v, CHUNK, 1.0)
    _fill_f32_1d(zeros_v, ROWS_PER_TILE, 0.0)
    pltpu.sync_copy(zeros_v, deg_sh.at[pl.ds(s * ROWS_PER_TILE, ROWS_PER_TILE)])
    plsc.subcore_barrier()
    pltpu.sync_copy(dstc_hbm.at[pl.ds(wid * CH_PER_TILE, CH_PER_TILE)], idxd_v)

    def body(j, carry):
      pltpu.sync_copy(ones_v, deg_sh.at[idxd_v.at[j]], add=True)
      return carry

    lax.fori_loop(0, CH_PER_TILE, body, None)
    plsc.subcore_barrier()
    pltpu.sync_copy(
        deg_sh.at[pl.ds(s * ROWS_PER_TILE, ROWS_PER_TILE)],
        out_hbm.at[pl.ds(c * NPAD + s * ROWS_PER_TILE, ROWS_PER_TILE)],
    )

  return k(dstc)


def _sc_scatter(gs, srcc, dstc):
  mesh = plsc.VectorSubcoreMesh(core_axis_name="c", subcore_axis_name="s")

  @functools.partial(
      pl.kernel,
      out_type=jax.ShapeDtypeStruct((2 * NPAD, D), jnp.float32),
      mesh=mesh,
      scratch_types=[
          pltpu.VMEM((CH_PER_TILE // 2, CHUNK), jnp.int32),
          pltpu.VMEM((CH_PER_TILE // 2, CHUNK), jnp.int32),
          [pltpu.VMEM((CHUNK, D), jnp.float32)] * 2,
          pltpu.VMEM_SHARED((NPAD, D), jnp.float32),
          [pltpu.SemaphoreType.DMA] * 2,
      ],
  )
  def k(gs_hbm, srcc_hbm, dstc_hbm, out_hbm, idxs_v, idxd_v, bufs, s_sh, sems):
    c = lax.axis_index("c")
    s = lax.axis_index("s")
    wid = s * 2 + c
    _zero_f32_block(bufs[0], CHUNK, D)
    zdescs = [
        pltpu.async_copy(
            bufs[0],
            s_sh.at[pl.ds(s * ROWS_PER_TILE + bb * CHUNK, CHUNK)],
            sems[0])
        for bb in range(ROWS_PER_TILE // CHUNK)
    ]
    half = CH_PER_TILE // 2
    base0 = wid * CH_PER_TILE
    pltpu.sync_copy(srcc_hbm.at[pl.ds(base0, half)], idxs_v)
    pltpu.sync_copy(dstc_hbm.at[pl.ds(base0, half)], idxd_v)
    for dsc in zdescs:
      dsc.wait()
    plsc.subcore_barrier()

    for p in range(2):
      if p == 1:
        base = wid * CH_PER_TILE + half
        pltpu.sync_copy(srcc_hbm.at[pl.ds(base, half)], idxs_v)
        pltpu.sync_copy(dstc_hbm.at[pl.ds(base, half)], idxd_v)

      @pl.loop(0, half, step=2)
      def _pipe(j):
        descs = [
            pltpu.async_copy(gs_hbm.at[idxs_v.at[j + t]], bufs[t], sems[t])
            for t in range(2)
        ]
        for t in range(2):
          descs[t].wait()
          pltpu.sync_copy(bufs[t], s_sh.at[idxd_v.at[j + t]], add=True)

    plsc.subcore_barrier()
    pltpu.sync_copy(
        s_sh.at[pl.ds(s * ROWS_PER_TILE, ROWS_PER_TILE)],
        out_hbm.at[pl.ds(c * NPAD + s * ROWS_PER_TILE, ROWS_PER_TILE)],
    )

  return k(gs, srcc, dstc)


def _tc_dis(d0, d1):
  def body(d0_ref, d1_ref, out_ref):
    out_ref[...] = lax.rsqrt(d0_ref[...] + d1_ref[...] + 1.0)

  return pl.pallas_call(
      body,
      out_shape=jax.ShapeDtypeStruct((NPAD // 128, 128), jnp.float32),
  )(d0, d1)


def _tc_first(x, W, dis_col):
  def body(x_ref, w_ref, dis_ref, out_ref):
    g = jnp.dot(x_ref[...], w_ref[...], preferred_element_type=jnp.float32)
    out_ref[...] = dis_ref[...] * g

  return pl.pallas_call(
      body,
      grid=(RB,),
      in_specs=[
          pl.BlockSpec((RBLK, D), lambda r: (r, 0)),
          pl.BlockSpec((D, D), lambda r: (0, 0)),
          pl.BlockSpec((RBLK, 1), lambda r: (r, 0)),
      ],
      out_specs=pl.BlockSpec((RBLK, D), lambda r: (r, 0)),
      out_shape=jax.ShapeDtypeStruct((N, D), jnp.float32),
  )(x, W, dis_col)


def _tc_mid(s0, s1, gsp, dis_col, bias, W):
  def body(s0_ref, s1_ref, gsp_ref, dis_ref, b_ref, w_ref, out_ref):
    agg = dis_ref[...] * (s0_ref[...] + s1_ref[...] + gsp_ref[...])
    h = jnp.maximum(agg + b_ref[...], 0.0)
    g = jnp.dot(h, w_ref[...], preferred_element_type=jnp.float32)
    out_ref[...] = dis_ref[...] * g

  return pl.pallas_call(
      body,
      grid=(RB,),
      in_specs=[
          pl.BlockSpec((RBLK, D), lambda r: (r, 0)),
          pl.BlockSpec((RBLK, D), lambda r: (r, 0)),
          pl.BlockSpec((RBLK, D), lambda r: (r, 0)),
          pl.BlockSpec((RBLK, 1), lambda r: (r, 0)),
          pl.BlockSpec((1, D), lambda r: (0, 0)),
          pl.BlockSpec((D, D), lambda r: (0, 0)),
      ],
      out_specs=pl.BlockSpec((RBLK, D), lambda r: (r, 0)),
      out_shape=jax.ShapeDtypeStruct((N, D), jnp.float32),
  )(s0, s1, gsp, dis_col, bias, W)


def _tc_final(s0, s1, gsp, dis_col, bias, batch3, Wl, bl):
  def body(s0_ref, s1_ref, gsp_ref, dis_ref, b_ref, bat_ref, wl_ref, bl_ref,
           out_ref, acc_ref, cnt_ref):
    r = pl.program_id(0)

    @pl.when(r == 0)
    def _():
      acc_ref[...] = jnp.zeros((B, D), jnp.float32)
      cnt_ref[...] = jnp.zeros((B, 128), jnp.float32)

    agg = dis_ref[...] * (s0_ref[...] + s1_ref[...] + gsp_ref[...])
    h = agg + b_ref[...]
    bat = bat_ref[0]
    gid = lax.broadcasted_iota(jnp.int32, (B, RBLK), 0)
    p = jnp.where(bat == gid, 1.0, 0.0)
    acc_ref[...] += jnp.dot(p, h, preferred_element_type=jnp.float32)
    cnt_ref[...] += jnp.broadcast_to(
        jnp.sum(p, axis=1, keepdims=True), (B, 128))

    @pl.when(r == RB - 1)
    def _():
      x_g = acc_ref[...] / jnp.maximum(cnt_ref[...], 1.0)
      logits = (
          jnp.dot(x_g, wl_ref[...], preferred_element_type=jnp.float32)
          + bl_ref[...])
      m = jnp.max(logits, axis=-1, keepdims=True)
      z = logits - m
      lse = jnp.log(jnp.sum(jnp.exp(z), axis=-1, keepdims=True))
      out_ref[...] = z - lse

  return pl.pallas_call(
      body,
      grid=(RB,),
      in_specs=[
          pl.BlockSpec((RBLK, D), lambda r: (r, 0)),
          pl.BlockSpec((RBLK, D), lambda r: (r, 0)),
          pl.BlockSpec((RBLK, D), lambda r: (r, 0)),
          pl.BlockSpec((RBLK, 1), lambda r: (r, 0)),
          pl.BlockSpec((1, D), lambda r: (0, 0)),
          pl.BlockSpec((1, 1, RBLK), lambda r: (r, 0, 0)),
          pl.BlockSpec((D, C), lambda r: (0, 0)),
          pl.BlockSpec((1, C), lambda r: (0, 0)),
      ],
      out_specs=pl.BlockSpec((B, C), lambda r: (0, 0)),
      out_shape=jax.ShapeDtypeStruct((B, C), jnp.float32),
      scratch_shapes=[
          pltpu.VMEM((B, D), jnp.float32),
          pltpu.VMEM((B, 128), jnp.float32),
      ],
  )(s0, s1, gsp, dis_col, bias, batch3, Wl, bl)


def kernel(x, edge_index, batch, W1, b1, W2, b2, W3, b3, Wl, bl):
  src = edge_index[0]
  dst = edge_index[1]
  pad = EPAD - E
  srcc = jnp.concatenate([src, jnp.zeros((pad,), jnp.int32)]).reshape(
      NCH, CHUNK)
  dstc = jnp.concatenate([dst, jnp.full((pad,), N, jnp.int32)]).reshape(
      NCH, CHUNK)

  degp = _sc_degree(dstc)
  d0 = degp[:NPAD].reshape(NPAD // 128, 128)
  d1 = degp[NPAD:].reshape(NPAD // 128, 128)
  dis_col = _tc_dis(d0, d1).reshape(NPAD)[:N].reshape(N, 1)

  b1r = b1.reshape(1, D)
  b2r = b2.reshape(1, D)
  b3r = b3.reshape(1, D)
  blr = bl.reshape(1, C)
  batch3 = batch.reshape(RB, 1, RBLK)

  gs1 = _tc_first(x, W1, dis_col)
  s = _sc_scatter(gs1, srcc, dstc)
  gs2 = _tc_mid(s[:N], s[NPAD:NPAD + N], gs1, dis_col, b1r, W2)
  s = _sc_scatter(gs2, srcc, dstc)
  gs3 = _tc_mid(s[:N], s[NPAD:NPAD + N], gs2, dis_col, b2r, W3)
  s = _sc_scatter(gs3, srcc, dstc)
  return _tc_final(s[:N], s[NPAD:NPAD + N], gs3, dis_col, b3r, batch3, Wl, blr)


# speedup vs baseline: 13.8844x; 2.9167x over previous
import functools

import jax
import jax.numpy as jnp
from jax import lax
from jax.experimental import pallas as pl
from jax.experimental.pallas import tpu as pltpu
from jax.experimental.pallas import tpu_sc as plsc

N = 10000
E = 320000
D = 128
C = 40
B = 64

NPAD = 10240
CHUNK = 128
NW = 32
EPAD = 327680
NCH = EPAD // CHUNK
CH_PER_TILE = NCH // NW
ROWS_PER_TILE = NPAD // 16
RB = 10
RBLK = N // RB


def _zero_f32_block(ref, rows, cols):
  groups = cols // 16

  def body(t, carry):
    i = t // groups
    g = t % groups
    ref[i, pl.ds(g * 16, 16)] = jnp.zeros((16,), jnp.float32)
    return carry

  lax.fori_loop(0, rows * groups, body, None)


def _fill_f32_1d(ref, n, value):
  def body(t, carry):
    ref[pl.ds(t * 16, 16)] = jnp.full((16,), value, jnp.float32)
    return carry

  lax.fori_loop(0, n // 16, body, None)


def _sc_degree(dstc):
  mesh = plsc.VectorSubcoreMesh(core_axis_name="c", subcore_axis_name="s")

  @functools.partial(
      pl.kernel,
      out_type=jax.ShapeDtypeStruct((2 * NPAD,), jnp.float32),
      mesh=mesh,
      scratch_types=[
          pltpu.VMEM((CH_PER_TILE, CHUNK), jnp.int32),
          pltpu.VMEM((CHUNK,), jnp.float32),
          pltpu.VMEM((ROWS_PER_TILE,), jnp.float32),
          pltpu.VMEM_SHARED((NPAD,), jnp.float32),
      ],
  )
  def k(dstc_hbm, out_hbm, idxd_v, ones_v, zeros_v, deg_sh):
    c = lax.axis_index("c")
    s = lax.axis_index("s")
    wid = s * 2 + c
    _fill_f32_1d(ones_v, CHUNK, 1.0)
    _fill_f32_1d(zeros_v, ROWS_PER_TILE, 0.0)
    pltpu.sync_copy(zeros_v, deg_sh.at[pl.ds(s * ROWS_PER_TILE, ROWS_PER_TILE)])
    plsc.subcore_barrier()
    pltpu.sync_copy(dstc_hbm.at[pl.ds(wid * CH_PER_TILE, CH_PER_TILE)], idxd_v)

    def body(j, carry):
      pltpu.sync_copy(ones_v, deg_sh.at[idxd_v.at[j]], add=True)
      return carry

    lax.fori_loop(0, CH_PER_TILE, body, None)
    plsc.subcore_barrier()
    pltpu.sync_copy(
        deg_sh.at[pl.ds(s * ROWS_PER_TILE, ROWS_PER_TILE)],
        out_hbm.at[pl.ds(c * NPAD + s * ROWS_PER_TILE, ROWS_PER_TILE)],
    )

  return k(dstc)


def _sc_scatter(gs, srcc, dstc):
  mesh = plsc.VectorSubcoreMesh(core_axis_name="c", subcore_axis_name="s")

  @functools.partial(
      pl.kernel,
      out_type=jax.ShapeDtypeStruct((2 * NPAD, D), jnp.float32),
      mesh=mesh,
      scratch_types=[
          pltpu.VMEM((CH_PER_TILE // 2, CHUNK), jnp.int32),
          pltpu.VMEM((CH_PER_TILE // 2, CHUNK), jnp.int32),
          [pltpu.VMEM((CHUNK, D), jnp.float32)] * 2,
          pltpu.VMEM_SHARED((NPAD, D), jnp.float32),
          [pltpu.SemaphoreType.DMA] * 2,
      ],
  )
  def k(gs_hbm, srcc_hbm, dstc_hbm, out_hbm, idxs_v, idxd_v, bufs, s_sh, sems):
    c = lax.axis_index("c")
    s = lax.axis_index("s")
    wid = s * 2 + c
    _zero_f32_block(bufs[0], CHUNK, D)
    zdescs = [
        pltpu.async_copy(
            bufs[0],
            s_sh.at[pl.ds(s * ROWS_PER_TILE + bb * CHUNK, CHUNK)],
            sems[0])
        for bb in range(ROWS_PER_TILE // CHUNK)
    ]
    half = CH_PER_TILE // 2
    base0 = wid * CH_PER_TILE
    pltpu.sync_copy(srcc_hbm.at[pl.ds(base0, half)], idxs_v)
    pltpu.sync_copy(dstc_hbm.at[pl.ds(base0, half)], idxd_v)
    for dsc in zdescs:
      dsc.wait()
    plsc.subcore_barrier()

    for p in range(2):
      if p == 1:
        base = wid * CH_PER_TILE + half
        pltpu.sync_copy(srcc_hbm.at[pl.ds(base, half)], idxs_v)
        pltpu.sync_copy(dstc_hbm.at[pl.ds(base, half)], idxd_v)

      @pl.loop(0, half, step=2)
      def _pipe(j):
        descs = [
            pltpu.async_copy(gs_hbm.at[idxs_v.at[j + t]], bufs[t], sems[t])
            for t in range(2)
        ]
        for t in range(2):
          descs[t].wait()
          pltpu.sync_copy(bufs[t], s_sh.at[idxd_v.at[j + t]], add=True)

    plsc.subcore_barrier()
    pltpu.sync_copy(
        s_sh.at[pl.ds(s * ROWS_PER_TILE, ROWS_PER_TILE)],
        out_hbm.at[pl.ds(c * NPAD + s * ROWS_PER_TILE, ROWS_PER_TILE)],
    )

  return k(gs, srcc, dstc)


def _tc_dis(d0, d1):
  def body(d0_ref, d1_ref, out_ref):
    out_ref[...] = lax.rsqrt(d0_ref[...] + d1_ref[...] + 1.0)

  return pl.pallas_call(
      body,
      out_shape=jax.ShapeDtypeStruct((NPAD // 128, 128), jnp.float32),
  )(d0, d1)


def _tc_first(x, W, dis_col):
  def body(x_ref, w_ref, dis_ref, out_ref):
    g = jnp.dot(x_ref[...], w_ref[...], preferred_element_type=jnp.float32)
    out_ref[...] = dis_ref[...] * g

  return pl.pallas_call(
      body,
      grid=(RB,),
      in_specs=[
          pl.BlockSpec((RBLK, D), lambda r: (r, 0)),
          pl.BlockSpec((D, D), lambda r: (0, 0)),
          pl.BlockSpec((RBLK, 1), lambda r: (r, 0)),
      ],
      out_specs=pl.BlockSpec((RBLK, D), lambda r: (r, 0)),
      out_shape=jax.ShapeDtypeStruct((N, D), jnp.float32),
  )(x, W, dis_col)


def _tc_mid(s0, s1, gsp, dis_col, bias, W):
  def body(s0_ref, s1_ref, gsp_ref, dis_ref, b_ref, w_ref, out_ref):
    agg = dis_ref[...] * (s0_ref[...] + s1_ref[...] + gsp_ref[...])
    h = jnp.maximum(agg + b_ref[...], 0.0)
    g = jnp.dot(h, w_ref[...], preferred_element_type=jnp.float32)
    out_ref[...] = dis_ref[...] * g

  return pl.pallas_call(
      body,
      grid=(RB,),
      in_specs=[
          pl.BlockSpec((RBLK, D), lambda r: (r, 0)),
          pl.BlockSpec((RBLK, D), lambda r: (r, 0)),
          pl.BlockSpec((RBLK, D), lambda r: (r, 0)),
          pl.BlockSpec((RBLK, 1), lambda r: (r, 0)),
          pl.BlockSpec((1, D), lambda r: (0, 0)),
          pl.BlockSpec((D, D), lambda r: (0, 0)),
      ],
      out_specs=pl.BlockSpec((RBLK, D), lambda r: (r, 0)),
      out_shape=jax.ShapeDtypeStruct((N, D), jnp.float32),
  )(s0, s1, gsp, dis_col, bias, W)


def _tc_final(s0, s1, gsp, dis_col, bias, batch3, Wl, bl):
  def body(s0_ref, s1_ref, gsp_ref, dis_ref, b_ref, bat_ref, wl_ref, bl_ref,
           out_ref, acc_ref, cnt_ref):
    r = pl.program_id(0)

    @pl.when(r == 0)
    def _():
      acc_ref[...] = jnp.zeros((B, D), jnp.float32)
      cnt_ref[...] = jnp.zeros((B, 128), jnp.float32)

    agg = dis_ref[...] * (s0_ref[...] + s1_ref[...] + gsp_ref[...])
    h = agg + b_ref[...]
    bat = bat_ref[0]
    gid = lax.broadcasted_iota(jnp.int32, (B, RBLK), 0)
    p = jnp.where(bat == gid, 1.0, 0.0)
    acc_ref[...] += jnp.dot(p, h, preferred_element_type=jnp.float32)
    cnt_ref[...] += jnp.broadcast_to(
        jnp.sum(p, axis=1, keepdims=True), (B, 128))

    @pl.when(r == RB - 1)
    def _():
      x_g = acc_ref[...] / jnp.maximum(cnt_ref[...], 1.0)
      logits = (
          jnp.dot(x_g, wl_ref[...], preferred_element_type=jnp.float32)
          + bl_ref[...])
      m = jnp.max(logits, axis=-1, keepdims=True)
      z = logits - m
      lse = jnp.log(jnp.sum(jnp.exp(z), axis=-1, keepdims=True))
      out_ref[...] = z - lse

  return pl.pallas_call(
      body,
      grid=(RB,),
      in_specs=[
          pl.BlockSpec((RBLK, D), lambda r: (r, 0)),
          pl.BlockSpec((RBLK, D), lambda r: (r, 0)),
          pl.BlockSpec((RBLK, D), lambda r: (r, 0)),
          pl.BlockSpec((RBLK, 1), lambda r: (r, 0)),
          pl.BlockSpec((1, D), lambda r: (0, 0)),
          pl.BlockSpec((1, 1, RBLK), lambda r: (r, 0, 0)),
          pl.BlockSpec((D, C), lambda r: (0, 0)),
          pl.BlockSpec((1, C), lambda r: (0, 0)),
      ],
      out_specs=pl.BlockSpec((B, C), lambda r: (0, 0)),
      out_shape=jax.ShapeDtypeStruct((B, C), jnp.float32),
      scratch_shapes=[
          pltpu.VMEM((B, D), jnp.float32),
          pltpu.VMEM((B, 128), jnp.float32),
      ],
  )(s0, s1, gsp, dis_col, bias, batch3, Wl, bl)


def kernel(x, edge_index, batch, W1, b1, W2, b2, W3, b3, Wl, bl):
  src = edge_index[0]
  dst = edge_index[1]
  pad = EPAD - E
  ar = jnp.arange(pad, dtype=jnp.int32)
  srcc = jnp.concatenate([src, ar % N]).reshape(NCH, CHUNK)
  dstc = jnp.concatenate([dst, N + (ar % (NPAD - N))]).reshape(NCH, CHUNK)

  degp = _sc_degree(dstc)
  d0 = degp[:NPAD].reshape(NPAD // 128, 128)
  d1 = degp[NPAD:].reshape(NPAD // 128, 128)
  dis_col = _tc_dis(d0, d1).reshape(NPAD)[:N].reshape(N, 1)

  b1r = b1.reshape(1, D)
  b2r = b2.reshape(1, D)
  b3r = b3.reshape(1, D)
  blr = bl.reshape(1, C)
  batch3 = batch.reshape(RB, 1, RBLK)

  gs1 = _tc_first(x, W1, dis_col)
  s = _sc_scatter(gs1, srcc, dstc)
  gs2 = _tc_mid(s[:N], s[NPAD:NPAD + N], gs1, dis_col, b1r, W2)
  s = _sc_scatter(gs2, srcc, dstc)
  gs3 = _tc_mid(s[:N], s[NPAD:NPAD + N], gs2, dis_col, b2r, W3)
  s = _sc_scatter(gs3, srcc, dstc)
  return _tc_final(s[:N], s[NPAD:NPAD + N], gs3, dis_col, b3r, batch3, Wl, blr)


# speedup vs baseline: 14.0625x; 1.0128x over previous
import functools

import jax
import jax.numpy as jnp
from jax import lax
from jax.experimental import pallas as pl
from jax.experimental.pallas import tpu as pltpu
from jax.experimental.pallas import tpu_sc as plsc

N = 10000
E = 320000
D = 128
C = 40
B = 64

NPAD = 10240
CHUNK = 128
NW = 32
EPAD = 327680
NCH = EPAD // CHUNK
CH_PER_TILE = NCH // NW
ROWS_PER_TILE = NPAD // 16
RB = 10
RBLK = N // RB


def _zero_f32_block(ref, rows, cols):
  groups = cols // 16

  def body(t, carry):
    i = t // groups
    g = t % groups
    ref[i, pl.ds(g * 16, 16)] = jnp.zeros((16,), jnp.float32)
    return carry

  lax.fori_loop(0, rows * groups, body, None)


def _fill_f32_1d(ref, n, value):
  def body(t, carry):
    ref[pl.ds(t * 16, 16)] = jnp.full((16,), value, jnp.float32)
    return carry

  lax.fori_loop(0, n // 16, body, None)


def _sc_degree(dstc):
  mesh = plsc.VectorSubcoreMesh(core_axis_name="c", subcore_axis_name="s")

  @functools.partial(
      pl.kernel,
      out_type=jax.ShapeDtypeStruct((2 * NPAD,), jnp.float32),
      mesh=mesh,
      scratch_types=[
          pltpu.VMEM((CH_PER_TILE, CHUNK), jnp.int32),
          pltpu.VMEM((CHUNK,), jnp.float32),
          pltpu.VMEM((ROWS_PER_TILE,), jnp.float32),
          pltpu.VMEM_SHARED((NPAD,), jnp.float32),
      ],
  )
  def k(dstc_hbm, out_hbm, idxd_v, ones_v, zeros_v, deg_sh):
    c = lax.axis_index("c")
    s = lax.axis_index("s")
    wid = s * 2 + c
    _fill_f32_1d(ones_v, CHUNK, 1.0)
    _fill_f32_1d(zeros_v, ROWS_PER_TILE, 0.0)
    pltpu.sync_copy(zeros_v, deg_sh.at[pl.ds(s * ROWS_PER_TILE, ROWS_PER_TILE)])
    plsc.subcore_barrier()
    pltpu.sync_copy(dstc_hbm.at[pl.ds(wid * CH_PER_TILE, CH_PER_TILE)], idxd_v)

    def body(j, carry):
      pltpu.sync_copy(ones_v, deg_sh.at[idxd_v.at[j]], add=True)
      return carry

    lax.fori_loop(0, CH_PER_TILE, body, None)
    plsc.subcore_barrier()
    pltpu.sync_copy(
        deg_sh.at[pl.ds(s * ROWS_PER_TILE, ROWS_PER_TILE)],
        out_hbm.at[pl.ds(c * NPAD + s * ROWS_PER_TILE, ROWS_PER_TILE)],
    )

  return k(dstc)


def _sc_scatter(gs, srcc, dstc):
  mesh = plsc.VectorSubcoreMesh(core_axis_name="c", subcore_axis_name="s")

  @functools.partial(
      pl.kernel,
      out_type=jax.ShapeDtypeStruct((2 * NPAD, D), jnp.float32),
      mesh=mesh,
      scratch_types=[
          pltpu.VMEM((CH_PER_TILE // 2, CHUNK), jnp.int32),
          pltpu.VMEM((CH_PER_TILE // 2, CHUNK), jnp.int32),
          [pltpu.VMEM((CHUNK, D), jnp.float32)] * 2,
          pltpu.VMEM_SHARED((NPAD, D), jnp.float32),
          [pltpu.SemaphoreType.DMA] * 4,
      ],
  )
  def k(gs_hbm, srcc_hbm, dstc_hbm, out_hbm, idxs_v, idxd_v, bufs, s_sh, sems):
    c = lax.axis_index("c")
    s = lax.axis_index("s")
    wid = s * 2 + c
    _zero_f32_block(bufs[0], CHUNK, D)
    zdescs = [
        pltpu.async_copy(
            bufs[0],
            s_sh.at[pl.ds(s * ROWS_PER_TILE + bb * CHUNK, CHUNK)],
            sems[0])
        for bb in range(ROWS_PER_TILE // CHUNK)
    ]
    half = CH_PER_TILE // 2
    base0 = wid * CH_PER_TILE
    pltpu.sync_copy(srcc_hbm.at[pl.ds(base0, half)], idxs_v)
    pltpu.sync_copy(dstc_hbm.at[pl.ds(base0, half)], idxd_v)
    for dsc in zdescs:
      dsc.wait()
    plsc.subcore_barrier()

    for p in range(2):
      if p == 1:
        base = wid * CH_PER_TILE + half
        pltpu.sync_copy(srcc_hbm.at[pl.ds(base, half)], idxs_v)
        pltpu.sync_copy(dstc_hbm.at[pl.ds(base, half)], idxd_v)

      @pl.loop(0, half, step=2)
      def _pipe(j):
        descs = [
            pltpu.async_copy(gs_hbm.at[idxs_v.at[j + t]], bufs[t], sems[t])
            for t in range(2)
        ]
        sdescs = []
        for t in range(2):
          descs[t].wait()
          sdescs.append(pltpu.async_copy(
              bufs[t], s_sh.at[idxd_v.at[j + t]], sems[2 + t], add=True))
        for sd in sdescs:
          sd.wait()

    plsc.subcore_barrier()
    pltpu.sync_copy(
        s_sh.at[pl.ds(s * ROWS_PER_TILE, ROWS_PER_TILE)],
        out_hbm.at[pl.ds(c * NPAD + s * ROWS_PER_TILE, ROWS_PER_TILE)],
    )

  return k(gs, srcc, dstc)


def _tc_dis(d0, d1):
  def body(d0_ref, d1_ref, out_ref):
    out_ref[...] = lax.rsqrt(d0_ref[...] + d1_ref[...] + 1.0)

  return pl.pallas_call(
      body,
      out_shape=jax.ShapeDtypeStruct((NPAD // 128, 128), jnp.float32),
  )(d0, d1)


def _tc_first(x, W, dis_col):
  def body(x_ref, w_ref, dis_ref, out_ref):
    g = jnp.dot(x_ref[...], w_ref[...], preferred_element_type=jnp.float32)
    out_ref[...] = dis_ref[...] * g

  return pl.pallas_call(
      body,
      grid=(RB,),
      in_specs=[
          pl.BlockSpec((RBLK, D), lambda r: (r, 0)),
          pl.BlockSpec((D, D), lambda r: (0, 0)),
          pl.BlockSpec((RBLK, 1), lambda r: (r, 0)),
      ],
      out_specs=pl.BlockSpec((RBLK, D), lambda r: (r, 0)),
      out_shape=jax.ShapeDtypeStruct((N, D), jnp.float32),
  )(x, W, dis_col)


def _tc_mid(s0, s1, gsp, dis_col, bias, W):
  def body(s0_ref, s1_ref, gsp_ref, dis_ref, b_ref, w_ref, out_ref):
    agg = dis_ref[...] * (s0_ref[...] + s1_ref[...] + gsp_ref[...])
    h = jnp.maximum(agg + b_ref[...], 0.0)
    g = jnp.dot(h, w_ref[...], preferred_element_type=jnp.float32)
    out_ref[...] = dis_ref[...] * g

  return pl.pallas_call(
      body,
      grid=(RB,),
      in_specs=[
          pl.BlockSpec((RBLK, D), lambda r: (r, 0)),
          pl.BlockSpec((RBLK, D), lambda r: (r, 0)),
          pl.BlockSpec((RBLK, D), lambda r: (r, 0)),
          pl.BlockSpec((RBLK, 1), lambda r: (r, 0)),
          pl.BlockSpec((1, D), lambda r: (0, 0)),
          pl.BlockSpec((D, D), lambda r: (0, 0)),
      ],
      out_specs=pl.BlockSpec((RBLK, D), lambda r: (r, 0)),
      out_shape=jax.ShapeDtypeStruct((N, D), jnp.float32),
  )(s0, s1, gsp, dis_col, bias, W)


def _tc_final(s0, s1, gsp, dis_col, bias, batch3, Wl, bl):
  def body(s0_ref, s1_ref, gsp_ref, dis_ref, b_ref, bat_ref, wl_ref, bl_ref,
           out_ref, acc_ref, cnt_ref):
    r = pl.program_id(0)

    @pl.when(r == 0)
    def _():
      acc_ref[...] = jnp.zeros((B, D), jnp.float32)
      cnt_ref[...] = jnp.zeros((B, 128), jnp.float32)

    agg = dis_ref[...] * (s0_ref[...] + s1_ref[...] + gsp_ref[...])
    h = agg + b_ref[...]
    bat = bat_ref[0]
    gid = lax.broadcasted_iota(jnp.int32, (B, RBLK), 0)
    p = jnp.where(bat == gid, 1.0, 0.0)
    acc_ref[...] += jnp.dot(p, h, preferred_element_type=jnp.float32)
    cnt_ref[...] += jnp.broadcast_to(
        jnp.sum(p, axis=1, keepdims=True), (B, 128))

    @pl.when(r == RB - 1)
    def _():
      x_g = acc_ref[...] / jnp.maximum(cnt_ref[...], 1.0)
      logits = (
          jnp.dot(x_g, wl_ref[...], preferred_element_type=jnp.float32)
          + bl_ref[...])
      m = jnp.max(logits, axis=-1, keepdims=True)
      z = logits - m
      lse = jnp.log(jnp.sum(jnp.exp(z), axis=-1, keepdims=True))
      out_ref[...] = z - lse

  return pl.pallas_call(
      body,
      grid=(RB,),
      in_specs=[
          pl.BlockSpec((RBLK, D), lambda r: (r, 0)),
          pl.BlockSpec((RBLK, D), lambda r: (r, 0)),
          pl.BlockSpec((RBLK, D), lambda r: (r, 0)),
          pl.BlockSpec((RBLK, 1), lambda r: (r, 0)),
          pl.BlockSpec((1, D), lambda r: (0, 0)),
          pl.BlockSpec((1, 1, RBLK), lambda r: (r, 0, 0)),
          pl.BlockSpec((D, C), lambda r: (0, 0)),
          pl.BlockSpec((1, C), lambda r: (0, 0)),
      ],
      out_specs=pl.BlockSpec((B, C), lambda r: (0, 0)),
      out_shape=jax.ShapeDtypeStruct((B, C), jnp.float32),
      scratch_shapes=[
          pltpu.VMEM((B, D), jnp.float32),
          pltpu.VMEM((B, 128), jnp.float32),
      ],
  )(s0, s1, gsp, dis_col, bias, batch3, Wl, bl)


def kernel(x, edge_index, batch, W1, b1, W2, b2, W3, b3, Wl, bl):
  src = edge_index[0]
  dst = edge_index[1]
  pad = EPAD - E
  ar = jnp.arange(pad, dtype=jnp.int32)
  srcc = jnp.concatenate([src, ar % N]).reshape(NCH, CHUNK)
  dstc = jnp.concatenate([dst, N + (ar % (NPAD - N))]).reshape(NCH, CHUNK)

  degp = _sc_degree(dstc)
  d0 = degp[:NPAD].reshape(NPAD // 128, 128)
  d1 = degp[NPAD:].reshape(NPAD // 128, 128)
  dis_col = _tc_dis(d0, d1).reshape(NPAD)[:N].reshape(N, 1)

  b1r = b1.reshape(1, D)
  b2r = b2.reshape(1, D)
  b3r = b3.reshape(1, D)
  blr = bl.reshape(1, C)
  batch3 = batch.reshape(RB, 1, RBLK)

  gs1 = _tc_first(x, W1, dis_col)
  s = _sc_scatter(gs1, srcc, dstc)
  gs2 = _tc_mid(s[:N], s[NPAD:NPAD + N], gs1, dis_col, b1r, W2)
  s = _sc_scatter(gs2, srcc, dstc)
  gs3 = _tc_mid(s[:N], s[NPAD:NPAD + N], gs2, dis_col, b2r, W3)
  s = _sc_scatter(gs3, srcc, dstc)
  return _tc_final(s[:N], s[NPAD:NPAD + N], gs3, dis_col, b3r, batch3, Wl, blr)
